# Initial kernel scaffold; baseline (speedup 1.0000x reference)
import jax
import jax.numpy as jnp
from jax.experimental import pallas as pl


def kernel(x, edge_index, W1, b1, W2, b2, W3, b3):
    raise NotImplementedError("write your pallas kernel here")


# speedup vs baseline: 4.3698x; 4.3698x over previous
import functools

import jax
import jax.numpy as jnp
from jax import lax
from jax.experimental import pallas as pl
from jax.experimental.pallas import tpu as pltpu
from jax.experimental.pallas import tpu_sc as plsc

N = 10000
E = 320000
D = 128
C = 40
CP = 128

NC = 2
NS = 16
NW = NC * NS

K = 128
EPW = 10240
EPAD = EPW * NW
NPAD = 10368
RPT = NPAD // NS
DUMP = 10240

ROWBLK = 1000


def _sc_mesh():
    return plsc.VectorSubcoreMesh(
        core_axis_name="c", subcore_axis_name="s", num_cores=NC, num_subcores=NS
    )


@functools.partial(
    pl.kernel,
    out_type=jax.ShapeDtypeStruct((NC, NPAD, D), jnp.float32),
    mesh=_sc_mesh(),
    scratch_types=[
        pltpu.VMEM((K,), jnp.int32),
        pltpu.VMEM((K, D), jnp.float32),
        pltpu.VMEM_SHARED((NPAD, D), jnp.float32),
    ],
)
def _sc_degree(dst_hbm, ones_hbm, zeros_hbm, out_hbm, dst_v, ones_v, acc_sh):
    cid = lax.axis_index("c")
    sid = lax.axis_index("s")
    wid = sid * NC + cid
    pltpu.sync_copy(ones_hbm, ones_v)
    pltpu.sync_copy(zeros_hbm, acc_sh.at[pl.ds(sid * RPT, RPT)])
    plsc.subcore_barrier()

    def body(j, _):
        pltpu.sync_copy(dst_hbm.at[pl.ds(wid * EPW + j * K, K)], dst_v)
        pltpu.sync_copy(ones_v, acc_sh.at[dst_v], add=True)
        return _

    lax.fori_loop(0, EPW // K, body, 0)
    plsc.subcore_barrier()
    pltpu.sync_copy(
        acc_sh.at[pl.ds(sid * RPT, RPT)], out_hbm.at[cid, pl.ds(sid * RPT, RPT)]
    )


def _make_segsum(width):
    @functools.partial(
        pl.kernel,
        out_type=jax.ShapeDtypeStruct((NC, NPAD, width), jnp.float32),
        mesh=_sc_mesh(),
        scratch_types=[
            pltpu.VMEM((K,), jnp.int32),
            pltpu.VMEM((K,), jnp.int32),
            pltpu.VMEM((K, width), jnp.float32),
            pltpu.VMEM_SHARED((NPAD, width), jnp.float32),
            pltpu.SemaphoreType.DMA,
        ],
    )
    def segsum(rows_hbm, src_hbm, dst_hbm, zeros_hbm, out_hbm,
               src_v, dst_v, rows_v, acc_sh, sem):
        cid = lax.axis_index("c")
        sid = lax.axis_index("s")
        wid = sid * NC + cid
        pltpu.sync_copy(zeros_hbm, acc_sh.at[pl.ds(sid * RPT, RPT)])
        plsc.subcore_barrier()

        def body(j, _):
            base = wid * EPW + j * K
            pltpu.sync_copy(src_hbm.at[pl.ds(base, K)], src_v)
            pltpu.sync_copy(dst_hbm.at[pl.ds(base, K)], dst_v)
            pltpu.async_copy(rows_hbm.at[src_v], rows_v, sem).wait()
            pltpu.sync_copy(rows_v, acc_sh.at[dst_v], add=True)
            return _

        lax.fori_loop(0, EPW // K, body, 0)
        plsc.subcore_barrier()
        pltpu.sync_copy(
            acc_sh.at[pl.ds(sid * RPT, RPT)], out_hbm.at[cid, pl.ds(sid * RPT, RPT)]
        )

    return segsum


_segsum_d = _make_segsum(D)
_segsum_c = _make_segsum(CP)


def _gelu(x):
    return 0.5 * x * (1.0 + lax.erf(x * 0.7071067811865476))


_row = lambda i: (i, 0)
_c00 = lambda i: (0, 0)


def _blk(shape, imap):
    return pl.BlockSpec(shape, imap)


def _tc_pre(d0, d1, x, w1):
    def body(d0_ref, d1_ref, x_ref, w_ref, h_ref, hs_ref, dinv_ref):
        deg = d0_ref[...] + d1_ref[...] + 2.0
        dinv = lax.rsqrt(deg)
        h = jnp.dot(x_ref[...], w_ref[...], preferred_element_type=jnp.float32)
        h_ref[...] = h
        hs_ref[...] = dinv * h
        dinv_ref[...] = dinv

    return pl.pallas_call(
        body,
        grid=(N // ROWBLK,),
        in_specs=[
            _blk((ROWBLK, 1), _row),
            _blk((ROWBLK, 1), _row),
            _blk((ROWBLK, D), _row),
            _blk((D, D), _c00),
        ],
        out_specs=[
            _blk((ROWBLK, D), _row),
            _blk((ROWBLK, D), _row),
            _blk((ROWBLK, 1), _row),
        ],
        out_shape=[
            jax.ShapeDtypeStruct((N, D), jnp.float32),
            jax.ShapeDtypeStruct((N, D), jnp.float32),
            jax.ShapeDtypeStruct((N, 1), jnp.float32),
        ],
    )(d0, d1, x, w1)


def _tc_mid(p, h, dinv, b, w, res=None):
    wout = w.shape[1]
    nres = 0 if res is None else 1

    def body(*refs):
        p0_ref, p1_ref, h_ref, dinv_ref, b_ref, w_ref = refs[:6]
        res_ref = refs[6] if nres else None
        a_ref, hn_ref, hsn_ref = refs[6 + nres:]
        dinv = dinv_ref[...]
        t = (dinv * (p0_ref[0] + p1_ref[0])
             + (2.0 * dinv * dinv) * h_ref[...] + b_ref[...])
        if nres:
            t = t + res_ref[...]
        a = _gelu(t)
        hn = jnp.dot(a, w_ref[...], preferred_element_type=jnp.float32)
        a_ref[...] = a
        hn_ref[...] = hn
        hsn_ref[...] = dinv * hn

    in_specs = [
        _blk((1, ROWBLK, D), lambda i: (0, i, 0)),
        _blk((1, ROWBLK, D), lambda i: (1, i, 0)),
        _blk((ROWBLK, D), _row),
        _blk((ROWBLK, 1), _row),
        _blk((1, D), _c00),
        _blk((D, wout), _c00),
    ]
    args = [p, p, h, dinv, b, w]
    if nres:
        in_specs.append(_blk((ROWBLK, D), _row))
        args.append(res)
    return pl.pallas_call(
        body,
        grid=(N // ROWBLK,),
        in_specs=in_specs,
        out_specs=[
            _blk((ROWBLK, D), _row),
            _blk((ROWBLK, wout), _row),
            _blk((ROWBLK, wout), _row),
        ],
        out_shape=[
            jax.ShapeDtypeStruct((N, D), jnp.float32),
            jax.ShapeDtypeStruct((N, wout), jnp.float32),
            jax.ShapeDtypeStruct((N, wout), jnp.float32),
        ],
    )(*args)


def _tc_final(p, h, dinv, b):
    def body(p0_ref, p1_ref, h_ref, dinv_ref, b_ref, o_ref):
        dinv = dinv_ref[...]
        o_ref[...] = (dinv * (p0_ref[0] + p1_ref[0])
                      + (2.0 * dinv * dinv) * h_ref[...] + b_ref[...])

    return pl.pallas_call(
        body,
        grid=(N // ROWBLK,),
        in_specs=[
            _blk((1, ROWBLK, CP), lambda i: (0, i, 0)),
            _blk((1, ROWBLK, CP), lambda i: (1, i, 0)),
            _blk((ROWBLK, CP), _row),
            _blk((ROWBLK, 1), _row),
            _blk((1, CP), _c00),
        ],
        out_specs=_blk((ROWBLK, CP), _row),
        out_shape=jax.ShapeDtypeStruct((N, CP), jnp.float32),
    )(p, p, h, dinv, b)


def kernel(x, edge_index, W1, b1, W2, b2, W3, b3):
    src = jnp.concatenate([edge_index[0], jnp.zeros((EPAD - E,), jnp.int32)])
    dst = jnp.concatenate(
        [edge_index[1], jnp.full((EPAD - E,), DUMP, jnp.int32)]
    )
    ones_d = jnp.ones((K, D), jnp.float32)
    zeros_d = jnp.zeros((RPT, D), jnp.float32)
    zeros_c = jnp.zeros((RPT, CP), jnp.float32)
    w3p = jnp.zeros((D, CP), jnp.float32).at[:, :C].set(W3)
    b3p = jnp.zeros((1, CP), jnp.float32).at[0, :C].set(b3)

    deg = _sc_degree(dst, ones_d, zeros_d)
    d0 = deg[0, :N, 0:1]
    d1 = deg[1, :N, 0:1]

    h1, hs1, dinv = _tc_pre(d0, d1, x, W1)
    p1 = _segsum_d(hs1, src, dst, zeros_d)
    a1, h2, hs2 = _tc_mid(p1[:, :N, :], h1, dinv, b1.reshape(1, D), W2)
    p2 = _segsum_d(hs2, src, dst, zeros_d)
    _, h3, hs3 = _tc_mid(p2[:, :N, :], h2, dinv, b2.reshape(1, D), w3p, res=a1)
    p3 = _segsum_c(hs3, src, dst, zeros_c)
    out = _tc_final(p3[:, :N, :], h3, dinv, b3p)
    return out[:, :C]


# speedup vs baseline: 5.4264x; 1.2418x over previous
import functools

import jax
import jax.numpy as jnp
from jax import lax
from jax.experimental import pallas as pl
from jax.experimental.pallas import tpu as pltpu
from jax.experimental.pallas import tpu_sc as plsc

N = 10000
E = 320000
D = 128
C = 40
CP = 128

NC = 2
NS = 16
NW = NC * NS

K = 128
EPW = 10240
NCH = EPW // K
EPAD = EPW * NW
NPAD = 10368
RPT = NPAD // NS
DUMP = 10240

ROWBLK = 1000


def _sc_mesh():
    return plsc.VectorSubcoreMesh(
        core_axis_name="c", subcore_axis_name="s", num_cores=NC, num_subcores=NS
    )


def _unpack_chunk(packed_v, j, sidx_v, didx_v):
    for t in range(K // 16):
        v = packed_v[j, pl.ds(t * 16, 16)]
        sidx_v[pl.ds(t * 16, 16)] = lax.bitwise_and(v, 16383)
        didx_v[pl.ds(t * 16, 16)] = lax.shift_right_logical(v, 14)


@functools.partial(
    pl.kernel,
    out_type=jax.ShapeDtypeStruct((NC, NPAD, D), jnp.float32),
    mesh=_sc_mesh(),
    scratch_types=[
        pltpu.VMEM((NCH, K), jnp.int32),
        pltpu.VMEM((K,), jnp.int32),
        pltpu.VMEM((K,), jnp.int32),
        pltpu.VMEM((K, D), jnp.float32),
        pltpu.VMEM_SHARED((NPAD, D), jnp.float32),
    ],
)
def _sc_degree(packed_hbm, ones_hbm, zeros_hbm, out_hbm,
               packed_v, sidx_v, didx_v, ones_v, acc_sh):
    cid = lax.axis_index("c")
    sid = lax.axis_index("s")
    wid = sid * NC + cid
    pltpu.sync_copy(packed_hbm.at[wid], packed_v)
    pltpu.sync_copy(ones_hbm, ones_v)
    pltpu.sync_copy(zeros_hbm, acc_sh.at[pl.ds(sid * RPT, RPT)])
    plsc.subcore_barrier()

    def body(j, _):
        _unpack_chunk(packed_v, j, sidx_v, didx_v)
        pltpu.sync_copy(ones_v, acc_sh.at[didx_v], add=True)
        return _

    lax.fori_loop(0, NCH, body, 0)
    plsc.subcore_barrier()
    pltpu.sync_copy(
        acc_sh.at[pl.ds(sid * RPT, RPT)], out_hbm.at[cid, pl.ds(sid * RPT, RPT)]
    )


def _make_segsum(width):
    @functools.partial(
        pl.kernel,
        out_type=jax.ShapeDtypeStruct((NC, NPAD, width), jnp.float32),
        mesh=_sc_mesh(),
        scratch_types=[
            pltpu.VMEM((NCH, K), jnp.int32),
            pltpu.VMEM((K,), jnp.int32),
            pltpu.VMEM((K,), jnp.int32),
            pltpu.VMEM((K,), jnp.int32),
            pltpu.VMEM((K,), jnp.int32),
            pltpu.VMEM((K, width), jnp.float32),
            pltpu.VMEM((K, width), jnp.float32),
            pltpu.VMEM_SHARED((NPAD, width), jnp.float32),
            pltpu.SemaphoreType.DMA,
            pltpu.SemaphoreType.DMA,
        ],
    )
    def segsum(rows_hbm, packed_hbm, zeros_hbm, out_hbm,
               packed_v, sidx0, sidx1, didx0, didx1, buf0, buf1,
               acc_sh, sem0, sem1):
        cid = lax.axis_index("c")
        sid = lax.axis_index("s")
        wid = sid * NC + cid
        pltpu.sync_copy(packed_hbm.at[wid], packed_v)
        pltpu.sync_copy(zeros_hbm, acc_sh.at[pl.ds(sid * RPT, RPT)])
        plsc.subcore_barrier()

        _unpack_chunk(packed_v, 0, sidx0, didx0)
        pltpu.async_copy(rows_hbm.at[sidx0], buf0, sem0)
        _unpack_chunk(packed_v, 1, sidx1, didx1)
        pltpu.async_copy(rows_hbm.at[sidx1], buf1, sem1)

        def body(i, _):
            j = 2 * i
            pltpu.make_async_copy(rows_hbm.at[sidx0], buf0, sem0).wait()
            pltpu.sync_copy(buf0, acc_sh.at[didx0], add=True)
            _unpack_chunk(packed_v, jnp.minimum(j + 2, NCH - 1), sidx0, didx0)
            pltpu.async_copy(rows_hbm.at[sidx0], buf0, sem0)
            pltpu.make_async_copy(rows_hbm.at[sidx1], buf1, sem1).wait()
            pltpu.sync_copy(buf1, acc_sh.at[didx1], add=True)
            _unpack_chunk(packed_v, jnp.minimum(j + 3, NCH - 1), sidx1, didx1)
            pltpu.async_copy(rows_hbm.at[sidx1], buf1, sem1)
            return _

        lax.fori_loop(0, NCH // 2, body, 0)
        pltpu.make_async_copy(rows_hbm.at[sidx0], buf0, sem0).wait()
        pltpu.make_async_copy(rows_hbm.at[sidx1], buf1, sem1).wait()
        plsc.subcore_barrier()
        pltpu.sync_copy(
            acc_sh.at[pl.ds(sid * RPT, RPT)], out_hbm.at[cid, pl.ds(sid * RPT, RPT)]
        )

    return segsum


_segsum_d = _make_segsum(D)
_segsum_c = _make_segsum(CP)


def _gelu(x):
    return 0.5 * x * (1.0 + lax.erf(x * 0.7071067811865476))


_row = lambda i: (i, 0)
_c00 = lambda i: (0, 0)


def _blk(shape, imap):
    return pl.BlockSpec(shape, imap)


def _tc_pre(d0, d1, x, w1):
    def body(d0_ref, d1_ref, x_ref, w_ref, h_ref, hs_ref, dinv_ref):
        deg = d0_ref[...] + d1_ref[...] + 2.0
        dinv = lax.rsqrt(deg)
        h = jnp.dot(x_ref[...], w_ref[...], preferred_element_type=jnp.float32)
        h_ref[...] = h
        hs_ref[...] = dinv * h
        dinv_ref[...] = dinv

    return pl.pallas_call(
        body,
        grid=(N // ROWBLK,),
        in_specs=[
            _blk((ROWBLK, 1), _row),
            _blk((ROWBLK, 1), _row),
            _blk((ROWBLK, D), _row),
            _blk((D, D), _c00),
        ],
        out_specs=[
            _blk((ROWBLK, D), _row),
            _blk((ROWBLK, D), _row),
            _blk((ROWBLK, 1), _row),
        ],
        out_shape=[
            jax.ShapeDtypeStruct((N, D), jnp.float32),
            jax.ShapeDtypeStruct((N, D), jnp.float32),
            jax.ShapeDtypeStruct((N, 1), jnp.float32),
        ],
    )(d0, d1, x, w1)


def _tc_mid(p, h, dinv, b, w, res=None):
    wout = w.shape[1]
    nres = 0 if res is None else 1

    def body(*refs):
        p0_ref, p1_ref, h_ref, dinv_ref, b_ref, w_ref = refs[:6]
        res_ref = refs[6] if nres else None
        a_ref, hn_ref, hsn_ref = refs[6 + nres:]
        dinv = dinv_ref[...]
        t = (dinv * (p0_ref[0] + p1_ref[0])
             + (2.0 * dinv * dinv) * h_ref[...] + b_ref[...])
        if nres:
            t = t + res_ref[...]
        a = _gelu(t)
        hn = jnp.dot(a, w_ref[...], preferred_element_type=jnp.float32)
        a_ref[...] = a
        hn_ref[...] = hn
        hsn_ref[...] = dinv * hn

    in_specs = [
        _blk((1, ROWBLK, D), lambda i: (0, i, 0)),
        _blk((1, ROWBLK, D), lambda i: (1, i, 0)),
        _blk((ROWBLK, D), _row),
        _blk((ROWBLK, 1), _row),
        _blk((1, D), _c00),
        _blk((D, wout), _c00),
    ]
    args = [p, p, h, dinv, b, w]
    if nres:
        in_specs.append(_blk((ROWBLK, D), _row))
        args.append(res)
    return pl.pallas_call(
        body,
        grid=(N // ROWBLK,),
        in_specs=in_specs,
        out_specs=[
            _blk((ROWBLK, D), _row),
            _blk((ROWBLK, wout), _row),
            _blk((ROWBLK, wout), _row),
        ],
        out_shape=[
            jax.ShapeDtypeStruct((N, D), jnp.float32),
            jax.ShapeDtypeStruct((N, wout), jnp.float32),
            jax.ShapeDtypeStruct((N, wout), jnp.float32),
        ],
    )(*args)


def _tc_final(p, h, dinv, b):
    def body(p0_ref, p1_ref, h_ref, dinv_ref, b_ref, o_ref):
        dinv = dinv_ref[...]
        o_ref[...] = (dinv * (p0_ref[0] + p1_ref[0])
                      + (2.0 * dinv * dinv) * h_ref[...] + b_ref[...])

    return pl.pallas_call(
        body,
        grid=(N // ROWBLK,),
        in_specs=[
            _blk((1, ROWBLK, CP), lambda i: (0, i, 0)),
            _blk((1, ROWBLK, CP), lambda i: (1, i, 0)),
            _blk((ROWBLK, CP), _row),
            _blk((ROWBLK, 1), _row),
            _blk((1, CP), _c00),
        ],
        out_specs=_blk((ROWBLK, CP), _row),
        out_shape=jax.ShapeDtypeStruct((N, CP), jnp.float32),
    )(p, p, h, dinv, b)


def kernel(x, edge_index, W1, b1, W2, b2, W3, b3):
    src = jnp.concatenate([edge_index[0], jnp.zeros((EPAD - E,), jnp.int32)])
    dst = jnp.concatenate(
        [edge_index[1], jnp.full((EPAD - E,), DUMP, jnp.int32)]
    )
    packed = jnp.bitwise_or(src, jnp.left_shift(dst, 14)).reshape(NW, NCH, K)
    ones_d = jnp.ones((K, D), jnp.float32)
    zeros_d = jnp.zeros((RPT, D), jnp.float32)
    zeros_c = jnp.zeros((RPT, CP), jnp.float32)
    w3p = jnp.zeros((D, CP), jnp.float32).at[:, :C].set(W3)
    b3p = jnp.zeros((1, CP), jnp.float32).at[0, :C].set(b3)

    deg = _sc_degree(packed, ones_d, zeros_d)
    d0 = deg[0, :N, 0:1]
    d1 = deg[1, :N, 0:1]

    h1, hs1, dinv = _tc_pre(d0, d1, x, W1)
    p1 = _segsum_d(hs1, packed, zeros_d)
    a1, h2, hs2 = _tc_mid(p1[:, :N, :], h1, dinv, b1.reshape(1, D), W2)
    p2 = _segsum_d(hs2, packed, zeros_d)
    _, h3, hs3 = _tc_mid(p2[:, :N, :], h2, dinv, b2.reshape(1, D), w3p, res=a1)
    p3 = _segsum_c(hs3, packed, zeros_c)
    out = _tc_final(p3[:, :N, :], h3, dinv, b3p)
    return out[:, :C]


# speedup vs baseline: 6.1705x; 1.1371x over previous
import functools

import jax
import jax.numpy as jnp
from jax import lax
from jax.experimental import pallas as pl
from jax.experimental.pallas import tpu as pltpu
from jax.experimental.pallas import tpu_sc as plsc

N = 10000
E = 320000
D = 128
C = 40
CP = 128

NC = 2
NS = 16
NW = NC * NS

K = 128
EPW = 10240
NCH = EPW // K
EPAD = EPW * NW
NPAD = 10368
RPT = NPAD // NS
DUMP = 10240

ROWBLK = 1000


def _sc_mesh():
    return plsc.VectorSubcoreMesh(
        core_axis_name="c", subcore_axis_name="s", num_cores=NC, num_subcores=NS
    )


def _unpack_chunk(packed_v, j, sidx_v, didx_v):
    for t in range(K // 16):
        v = packed_v[j, pl.ds(t * 16, 16)]
        sidx_v[pl.ds(t * 16, 16)] = lax.bitwise_and(v, 16383)
        didx_v[pl.ds(t * 16, 16)] = lax.shift_right_logical(v, 14)


@functools.partial(
    pl.kernel,
    out_type=jax.ShapeDtypeStruct((NC, NPAD, D), jnp.float32),
    mesh=_sc_mesh(),
    scratch_types=[
        pltpu.VMEM((NCH, K), jnp.int32),
        pltpu.VMEM((K,), jnp.int32),
        pltpu.VMEM((K,), jnp.int32),
        pltpu.VMEM((K, D), jnp.float32),
        pltpu.VMEM_SHARED((NPAD, D), jnp.float32),
    ],
)
def _sc_degree(packed_hbm, ones_hbm, zeros_hbm, out_hbm,
               packed_v, sidx_v, didx_v, ones_v, acc_sh):
    cid = lax.axis_index("c")
    sid = lax.axis_index("s")
    wid = sid * NC + cid
    pltpu.sync_copy(packed_hbm.at[wid], packed_v)
    pltpu.sync_copy(ones_hbm, ones_v)
    pltpu.sync_copy(zeros_hbm, acc_sh.at[pl.ds(sid * RPT, RPT)])
    plsc.subcore_barrier()

    def body(j, _):
        _unpack_chunk(packed_v, j, sidx_v, didx_v)
        pltpu.sync_copy(ones_v, acc_sh.at[didx_v], add=True)
        return _

    lax.fori_loop(0, NCH, body, 0)
    plsc.subcore_barrier()
    pltpu.sync_copy(
        acc_sh.at[pl.ds(sid * RPT, RPT)], out_hbm.at[cid, pl.ds(sid * RPT, RPT)]
    )


def _make_segsum(width):
    @functools.partial(
        pl.kernel,
        out_type=jax.ShapeDtypeStruct((NC, NPAD, width), jnp.float32),
        mesh=_sc_mesh(),
        scratch_types=[
            pltpu.VMEM((NCH, K), jnp.int32),
            pltpu.VMEM((K,), jnp.int32),
            pltpu.VMEM((K,), jnp.int32),
            pltpu.VMEM((K,), jnp.int32),
            pltpu.VMEM((K,), jnp.int32),
            pltpu.VMEM((K, width), jnp.float32),
            pltpu.VMEM((K, width), jnp.float32),
            pltpu.VMEM_SHARED((NPAD, width), jnp.float32),
            pltpu.SemaphoreType.DMA,
            pltpu.SemaphoreType.DMA,
        ],
    )
    def segsum(rows_hbm, packed_hbm, zeros_hbm, out_hbm,
               packed_v, sidx0, sidx1, didx0, didx1, buf0, buf1,
               acc_sh, sem0, sem1):
        cid = lax.axis_index("c")
        sid = lax.axis_index("s")
        wid = sid * NC + cid
        pltpu.sync_copy(packed_hbm.at[wid], packed_v)
        pltpu.sync_copy(zeros_hbm, acc_sh.at[pl.ds(sid * RPT, RPT)])
        plsc.subcore_barrier()

        _unpack_chunk(packed_v, 0, sidx0, didx0)
        pltpu.async_copy(rows_hbm.at[sidx0], buf0, sem0)
        _unpack_chunk(packed_v, 1, sidx1, didx1)
        pltpu.async_copy(rows_hbm.at[sidx1], buf1, sem1)

        def body(i, _):
            j = 2 * i
            pltpu.make_async_copy(rows_hbm.at[sidx0], buf0, sem0).wait()
            pltpu.sync_copy(buf0, acc_sh.at[didx0], add=True)
            _unpack_chunk(packed_v, jnp.minimum(j + 2, NCH - 1), sidx0, didx0)
            pltpu.async_copy(rows_hbm.at[sidx0], buf0, sem0)
            pltpu.make_async_copy(rows_hbm.at[sidx1], buf1, sem1).wait()
            pltpu.sync_copy(buf1, acc_sh.at[didx1], add=True)
            _unpack_chunk(packed_v, jnp.minimum(j + 3, NCH - 1), sidx1, didx1)
            pltpu.async_copy(rows_hbm.at[sidx1], buf1, sem1)
            return _

        lax.fori_loop(0, NCH // 2, body, 0)
        pltpu.make_async_copy(rows_hbm.at[sidx0], buf0, sem0).wait()
        pltpu.make_async_copy(rows_hbm.at[sidx1], buf1, sem1).wait()
        plsc.subcore_barrier()
        pltpu.sync_copy(
            acc_sh.at[pl.ds(sid * RPT, RPT)], out_hbm.at[cid, pl.ds(sid * RPT, RPT)]
        )

    return segsum


_segsum_d = _make_segsum(D)
_segsum_c = _make_segsum(CP)


def _gelu(x):
    return 0.5 * x * (1.0 + lax.erf(x * 0.7071067811865476))


_row = lambda i: (i, 0)
_c00 = lambda i: (0, 0)


def _blk(shape, imap):
    return pl.BlockSpec(shape, imap)


def _tc_pre(deg2, x, w1):
    def body(d0_ref, d1_ref, x_ref, w_ref, h_ref, hs_ref, dinv_ref):
        deg = d0_ref[0, :, 0:1] + d1_ref[0, :, 0:1] + 2.0
        dinv = lax.rsqrt(deg)
        h = jnp.dot(x_ref[...], w_ref[...], preferred_element_type=jnp.float32)
        h_ref[...] = h
        hs_ref[...] = dinv * h
        dinv_ref[...] = dinv

    return pl.pallas_call(
        body,
        grid=(N // ROWBLK,),
        in_specs=[
            _blk((1, ROWBLK, D), lambda i: (0, i, 0)),
            _blk((1, ROWBLK, D), lambda i: (1, i, 0)),
            _blk((ROWBLK, D), _row),
            _blk((D, D), _c00),
        ],
        out_specs=[
            _blk((ROWBLK, D), _row),
            _blk((ROWBLK, D), _row),
            _blk((ROWBLK, 1), _row),
        ],
        out_shape=[
            jax.ShapeDtypeStruct((N, D), jnp.float32),
            jax.ShapeDtypeStruct((N, D), jnp.float32),
            jax.ShapeDtypeStruct((N, 1), jnp.float32),
        ],
    )(deg2, deg2, x, w1)


def _tc_mid(p, h, dinv, b, w, res=None):
    wout = w.shape[1]
    nres = 0 if res is None else 1

    def body(*refs):
        p0_ref, p1_ref, h_ref, dinv_ref, b_ref, w_ref = refs[:6]
        res_ref = refs[6] if nres else None
        a_ref, hn_ref, hsn_ref = refs[6 + nres:]
        dinv = dinv_ref[...]
        t = (dinv * (p0_ref[0] + p1_ref[0])
             + (2.0 * dinv * dinv) * h_ref[...] + b_ref[...])
        if nres:
            t = t + res_ref[...]
        a = _gelu(t)
        hn = jnp.dot(a, w_ref[...], preferred_element_type=jnp.float32)
        a_ref[...] = a
        hn_ref[...] = hn
        hsn_ref[...] = dinv * hn

    in_specs = [
        _blk((1, ROWBLK, D), lambda i: (0, i, 0)),
        _blk((1, ROWBLK, D), lambda i: (1, i, 0)),
        _blk((ROWBLK, D), _row),
        _blk((ROWBLK, 1), _row),
        _blk((1, D), _c00),
        _blk((D, wout), _c00),
    ]
    args = [p, p, h, dinv, b, w]
    if nres:
        in_specs.append(_blk((ROWBLK, D), _row))
        args.append(res)
    return pl.pallas_call(
        body,
        grid=(N // ROWBLK,),
        in_specs=in_specs,
        out_specs=[
            _blk((ROWBLK, D), _row),
            _blk((ROWBLK, wout), _row),
            _blk((ROWBLK, wout), _row),
        ],
        out_shape=[
            jax.ShapeDtypeStruct((N, D), jnp.float32),
            jax.ShapeDtypeStruct((N, wout), jnp.float32),
            jax.ShapeDtypeStruct((N, wout), jnp.float32),
        ],
    )(*args)


def _tc_final(p, h, dinv, b):
    def body(p0_ref, p1_ref, h_ref, dinv_ref, b_ref, o_ref):
        dinv = dinv_ref[...]
        o_ref[...] = (dinv * (p0_ref[0] + p1_ref[0])
                      + (2.0 * dinv * dinv) * h_ref[...] + b_ref[...])

    return pl.pallas_call(
        body,
        grid=(N // ROWBLK,),
        in_specs=[
            _blk((1, ROWBLK, CP), lambda i: (0, i, 0)),
            _blk((1, ROWBLK, CP), lambda i: (1, i, 0)),
            _blk((ROWBLK, CP), _row),
            _blk((ROWBLK, 1), _row),
            _blk((1, CP), _c00),
        ],
        out_specs=_blk((ROWBLK, CP), _row),
        out_shape=jax.ShapeDtypeStruct((N, CP), jnp.float32),
    )(p, p, h, dinv, b)


def kernel(x, edge_index, W1, b1, W2, b2, W3, b3):
    src = jnp.concatenate([edge_index[0], jnp.zeros((EPAD - E,), jnp.int32)])
    dst = jnp.concatenate(
        [edge_index[1], DUMP + (jnp.arange(EPAD - E, dtype=jnp.int32) % 128)]
    )
    packed = jnp.bitwise_or(src, jnp.left_shift(dst, 14)).reshape(NW, NCH, K)
    ones_d = jnp.ones((K, D), jnp.float32)
    zeros_d = jnp.zeros((RPT, D), jnp.float32)
    zeros_c = jnp.zeros((RPT, CP), jnp.float32)
    w3p = jnp.zeros((D, CP), jnp.float32).at[:, :C].set(W3)
    b3p = jnp.zeros((1, CP), jnp.float32).at[0, :C].set(b3)

    deg = _sc_degree(packed, ones_d, zeros_d)

    h1, hs1, dinv = _tc_pre(deg, x, W1)
    p1 = _segsum_d(hs1, packed, zeros_d)
    a1, h2, hs2 = _tc_mid(p1, h1, dinv, b1.reshape(1, D), W2)
    p2 = _segsum_d(hs2, packed, zeros_d)
    _, h3, hs3 = _tc_mid(p2, h2, dinv, b2.reshape(1, D), w3p, res=a1)
    p3 = _segsum_c(hs3, packed, zeros_c)
    out = _tc_final(p3, h3, dinv, b3p)
    return out[:, :C]


# speedup vs baseline: 19.4217x; 3.1475x over previous
import functools

import jax
import jax.numpy as jnp
from jax import lax
from jax.experimental import pallas as pl
from jax.experimental.pallas import tpu as pltpu
from jax.experimental.pallas import tpu_sc as plsc

N = 10000
E = 320000
D = 128
C = 40
CP = 128

NC = 2
NS = 16
NW = NC * NS

K = 128
EPW = 10240
NCH = EPW // K
EPAD = EPW * NW
NPAD = 10368
RPT = NPAD // NS
DUMP = 10240

ROWBLK = 1000


def _sc_mesh():
    return plsc.VectorSubcoreMesh(
        core_axis_name="c", subcore_axis_name="s", num_cores=NC, num_subcores=NS
    )


def _unpack_chunk(packed_v, j, sidx_v, didx_v):
    for t in range(K // 16):
        v = packed_v[j, pl.ds(t * 16, 16)]
        sidx_v[pl.ds(t * 16, 16)] = lax.bitwise_and(v, 16383)
        didx_v[pl.ds(t * 16, 16)] = lax.shift_right_logical(v, 14)


@functools.partial(
    pl.kernel,
    out_type=jax.ShapeDtypeStruct((NC, NPAD, D), jnp.float32),
    mesh=_sc_mesh(),
    scratch_types=[
        pltpu.VMEM((NCH, K), jnp.int32),
        pltpu.VMEM((K,), jnp.int32),
        pltpu.VMEM((K,), jnp.int32),
        pltpu.VMEM((K, D), jnp.float32),
        pltpu.VMEM_SHARED((NPAD, D), jnp.float32),
    ],
)
def _sc_degree(packed_hbm, ones_hbm, zeros_hbm, out_hbm,
               packed_v, sidx_v, didx_v, ones_v, acc_sh):
    cid = lax.axis_index("c")
    sid = lax.axis_index("s")
    wid = sid * NC + cid
    pltpu.sync_copy(packed_hbm.at[wid], packed_v)
    pltpu.sync_copy(ones_hbm, ones_v)
    pltpu.sync_copy(zeros_hbm, acc_sh.at[pl.ds(sid * RPT, RPT)])
    plsc.subcore_barrier()

    def body(j, _):
        _unpack_chunk(packed_v, j, sidx_v, didx_v)
        pltpu.sync_copy(ones_v, acc_sh.at[didx_v], add=True)
        return _

    lax.fori_loop(0, NCH, body, 0)
    plsc.subcore_barrier()
    pltpu.sync_copy(
        acc_sh.at[pl.ds(sid * RPT, RPT)], out_hbm.at[cid, pl.ds(sid * RPT, RPT)]
    )


def _make_segsum(width):
    @functools.partial(
        pl.kernel,
        out_type=jax.ShapeDtypeStruct((NC, NPAD, width), jnp.float32),
        mesh=_sc_mesh(),
        scratch_types=[
            pltpu.VMEM((NCH, K), jnp.int32),
            pltpu.VMEM((K,), jnp.int32),
            pltpu.VMEM((K,), jnp.int32),
            pltpu.VMEM((K,), jnp.int32),
            pltpu.VMEM((K,), jnp.int32),
            pltpu.VMEM((K, width), jnp.float32),
            pltpu.VMEM((K, width), jnp.float32),
            pltpu.VMEM_SHARED((NPAD, width), jnp.float32),
            pltpu.SemaphoreType.DMA,
            pltpu.SemaphoreType.DMA,
        ],
    )
    def segsum(rows_hbm, packed_hbm, zeros_hbm, out_hbm,
               packed_v, sidx0, sidx1, didx0, didx1, buf0, buf1,
               acc_sh, sem0, sem1):
        cid = lax.axis_index("c")
        sid = lax.axis_index("s")
        wid = sid * NC + cid
        pltpu.sync_copy(packed_hbm.at[wid], packed_v)
        pltpu.sync_copy(zeros_hbm, acc_sh.at[pl.ds(sid * RPT, RPT)])
        plsc.subcore_barrier()

        _unpack_chunk(packed_v, 0, sidx0, didx0)
        pltpu.async_copy(rows_hbm.at[sidx0], buf0, sem0)
        _unpack_chunk(packed_v, 1, sidx1, didx1)
        pltpu.async_copy(rows_hbm.at[sidx1], buf1, sem1)

        def body(i, _):
            j = 2 * i
            pltpu.make_async_copy(rows_hbm.at[sidx0], buf0, sem0).wait()
            pltpu.sync_copy(buf0, acc_sh.at[didx0], add=True)
            _unpack_chunk(packed_v, jnp.minimum(j + 2, NCH - 1), sidx0, didx0)
            pltpu.async_copy(rows_hbm.at[sidx0], buf0, sem0)
            pltpu.make_async_copy(rows_hbm.at[sidx1], buf1, sem1).wait()
            pltpu.sync_copy(buf1, acc_sh.at[didx1], add=True)
            _unpack_chunk(packed_v, jnp.minimum(j + 3, NCH - 1), sidx1, didx1)
            pltpu.async_copy(rows_hbm.at[sidx1], buf1, sem1)
            return _

        lax.fori_loop(0, NCH // 2, body, 0)
        pltpu.make_async_copy(rows_hbm.at[sidx0], buf0, sem0).wait()
        pltpu.make_async_copy(rows_hbm.at[sidx1], buf1, sem1).wait()
        plsc.subcore_barrier()
        pltpu.sync_copy(
            acc_sh.at[pl.ds(sid * RPT, RPT)], out_hbm.at[cid, pl.ds(sid * RPT, RPT)]
        )

    return segsum


_segsum_d = _make_segsum(D)
_segsum_c = _make_segsum(CP)


def _gelu(x):
    return 0.5 * x * (1.0 + lax.erf(x * 0.7071067811865476))


_row = lambda i: (i, 0)
_c00 = lambda i: (0, 0)


def _blk(shape, imap):
    return pl.BlockSpec(shape, imap)


def _tc_pre(deg2, x, w1):
    def body(d0_ref, d1_ref, x_ref, w_ref, h_ref, hs_ref, dinv_ref):
        deg = d0_ref[0, :, 0:1] + d1_ref[0, :, 0:1] + 2.0
        dinv = lax.rsqrt(deg)
        h = jnp.dot(x_ref[...], w_ref[...], preferred_element_type=jnp.float32)
        h_ref[...] = h
        hs_ref[...] = dinv * h
        dinv_ref[...] = dinv

    return pl.pallas_call(
        body,
        grid=(N // ROWBLK,),
        in_specs=[
            _blk((1, ROWBLK, D), lambda i: (0, i, 0)),
            _blk((1, ROWBLK, D), lambda i: (1, i, 0)),
            _blk((ROWBLK, D), _row),
            _blk((D, D), _c00),
        ],
        out_specs=[
            _blk((ROWBLK, D), _row),
            _blk((ROWBLK, D), _row),
            _blk((ROWBLK, 1), _row),
        ],
        out_shape=[
            jax.ShapeDtypeStruct((N, D), jnp.float32),
            jax.ShapeDtypeStruct((N, D), jnp.float32),
            jax.ShapeDtypeStruct((N, 1), jnp.float32),
        ],
    )(deg2, deg2, x, w1)


def _tc_mid(p, h, dinv, b, w, res=None):
    wout = w.shape[1]
    nres = 0 if res is None else 1

    def body(*refs):
        p0_ref, p1_ref, h_ref, dinv_ref, b_ref, w_ref = refs[:6]
        res_ref = refs[6] if nres else None
        a_ref, hn_ref, hsn_ref = refs[6 + nres:]
        dinv = dinv_ref[...]
        t = (dinv * (p0_ref[0] + p1_ref[0])
             + (2.0 * dinv * dinv) * h_ref[...] + b_ref[...])
        if nres:
            t = t + res_ref[...]
        a = _gelu(t)
        hn = jnp.dot(a, w_ref[...], preferred_element_type=jnp.float32)
        a_ref[...] = a
        hn_ref[...] = hn
        hsn_ref[...] = dinv * hn

    in_specs = [
        _blk((1, ROWBLK, D), lambda i: (0, i, 0)),
        _blk((1, ROWBLK, D), lambda i: (1, i, 0)),
        _blk((ROWBLK, D), _row),
        _blk((ROWBLK, 1), _row),
        _blk((1, D), _c00),
        _blk((D, wout), _c00),
    ]
    args = [p, p, h, dinv, b, w]
    if nres:
        in_specs.append(_blk((ROWBLK, D), _row))
        args.append(res)
    return pl.pallas_call(
        body,
        grid=(N // ROWBLK,),
        in_specs=in_specs,
        out_specs=[
            _blk((ROWBLK, D), _row),
            _blk((ROWBLK, wout), _row),
            _blk((ROWBLK, wout), _row),
        ],
        out_shape=[
            jax.ShapeDtypeStruct((N, D), jnp.float32),
            jax.ShapeDtypeStruct((N, wout), jnp.float32),
            jax.ShapeDtypeStruct((N, wout), jnp.float32),
        ],
    )(*args)


def _tc_final(p, h, dinv, b):
    def body(p0_ref, p1_ref, h_ref, dinv_ref, b_ref, o_ref):
        dinv = dinv_ref[...]
        o_ref[...] = (dinv * (p0_ref[0] + p1_ref[0])
                      + (2.0 * dinv * dinv) * h_ref[...] + b_ref[...])

    return pl.pallas_call(
        body,
        grid=(N // ROWBLK,),
        in_specs=[
            _blk((1, ROWBLK, CP), lambda i: (0, i, 0)),
            _blk((1, ROWBLK, CP), lambda i: (1, i, 0)),
            _blk((ROWBLK, CP), _row),
            _blk((ROWBLK, 1), _row),
            _blk((1, CP), _c00),
        ],
        out_specs=_blk((ROWBLK, CP), _row),
        out_shape=jax.ShapeDtypeStruct((N, CP), jnp.float32),
    )(p, p, h, dinv, b)


def kernel(x, edge_index, W1, b1, W2, b2, W3, b3):
    pad_iota = jnp.arange(EPAD - E, dtype=jnp.int32)
    src = jnp.concatenate([edge_index[0], pad_iota % 4096])
    dst = jnp.concatenate([edge_index[1], DUMP + (pad_iota % 128)])
    packed = jnp.bitwise_or(src, jnp.left_shift(dst, 14)).reshape(NW, NCH, K)
    ones_d = jnp.ones((K, D), jnp.float32)
    zeros_d = jnp.zeros((RPT, D), jnp.float32)
    zeros_c = jnp.zeros((RPT, CP), jnp.float32)
    w3p = jnp.zeros((D, CP), jnp.float32).at[:, :C].set(W3)
    b3p = jnp.zeros((1, CP), jnp.float32).at[0, :C].set(b3)

    deg = _sc_degree(packed, ones_d, zeros_d)

    h1, hs1, dinv = _tc_pre(deg, x, W1)
    p1 = _segsum_d(hs1, packed, zeros_d)
    a1, h2, hs2 = _tc_mid(p1, h1, dinv, b1.reshape(1, D), W2)
    p2 = _segsum_d(hs2, packed, zeros_d)
    _, h3, hs3 = _tc_mid(p2, h2, dinv, b2.reshape(1, D), w3p, res=a1)
    p3 = _segsum_c(hs3, packed, zeros_c)
    out = _tc_final(p3, h3, dinv, b3p)
    return out[:, :C]
